# baseline (speedup 1.0000x reference)
import functools
import math

import jax
import jax.numpy as jnp
from jax import lax
from jax.experimental import pallas as pl
from jax.experimental.pallas import tpu as pltpu
from jax.experimental.pallas import tpu_sc as plsc

D_EMBED = 32
VOCAB = 1000000
BATCH = 16384
HIST = 20
SCALE = math.sqrt(D_EMBED)

NC, NS = 2, 16
NW = NC * NS
B = BATCH * HIST
IDX_ROWS = B // 128
ROWS_PER_W = IDX_ROWS // NW
CHUNK_ROWS = 8
CHUNK = CHUNK_ROWS * 128
NCHUNK = ROWS_PER_W // CHUNK_ROWS

CSTRIDE = 1 << 18
TCB = 1024
TGRID = CSTRIDE // TCB
LAST_BLK = (VOCAB - 1) // TCB


def _relayout_body(t0, t1, t2, t3, o_ref):
    eye_s = jnp.float32(SCALE) * jnp.eye(D_EMBED, dtype=jnp.float32)
    for c, t in enumerate((t0, t1, t2, t3)):
        o_ref[:, c * D_EMBED:(c + 1) * D_EMBED] = lax.dot_general(
            t[...], eye_s, (((0,), (0,)), ((), ())),
            preferred_element_type=jnp.float32)


def _relayout(table_t):
    def imap(c):
        return lambda i: (0, jnp.minimum(c * TGRID + i, LAST_BLK))
    return pl.pallas_call(
        _relayout_body,
        grid=(TGRID,),
        in_specs=[pl.BlockSpec((D_EMBED, TCB), imap(c)) for c in range(4)],
        out_specs=pl.BlockSpec((TCB, 4 * D_EMBED), lambda i: (i, 0)),
        out_shape=jax.ShapeDtypeStruct((CSTRIDE, 4 * D_EMBED), jnp.float32),
    )(table_t, table_t, table_t, table_t)


def _gather_body(x_hbm, table_hbm, out_hbm, idx_v, rows0, rows1, sg0, sg1,
                 ss0, ss1):
    wid = lax.axis_index("s") * NC + lax.axis_index("c")
    row_base = wid * ROWS_PER_W
    out_base = wid * ROWS_PER_W * 128

    pltpu.sync_copy(x_hbm.at[pl.ds(row_base, ROWS_PER_W)], idx_v)

    def fire(g, buf, sem):
        for j in range(CHUNK_ROWS):
            pltpu.async_copy(
                table_hbm.at[idx_v.at[g * CHUNK_ROWS + j]],
                buf.at[pl.ds(j * 128, 128)],
                sem,
            )

    def drain_gathers(buf, sem):
        for j in range(CHUNK_ROWS):
            pltpu.make_async_copy(
                table_hbm.at[idx_v.at[j]],
                buf.at[pl.ds(j * 128, 128)],
                sem,
            ).wait()

    def out_at(g):
        return out_hbm.at[pl.ds(out_base + g * CHUNK, CHUNK)]

    def store(g, buf, sem):
        pltpu.async_copy(buf, out_at(g), sem)

    def drain_store(g, buf, sem):
        pltpu.make_async_copy(buf, out_at(g), sem).wait()

    fire(0, rows0, sg0)
    fire(1, rows1, sg1)

    @pl.loop(0, NCHUNK // 2 - 1)
    def _pair(i):
        g = 2 * i
        drain_gathers(rows0, sg0)
        store(g, rows0, ss0)
        drain_gathers(rows1, sg1)
        store(g + 1, rows1, ss1)
        drain_store(g, rows0, ss0)
        fire(g + 2, rows0, sg0)
        drain_store(g + 1, rows1, ss1)
        fire(g + 3, rows1, sg1)

    last = NCHUNK - 2
    drain_gathers(rows0, sg0)
    store(last, rows0, ss0)
    drain_gathers(rows1, sg1)
    store(last + 1, rows1, ss1)
    drain_store(last, rows0, ss0)
    drain_store(last + 1, rows1, ss1)


def _gather(x2d, table_lin):
    mesh = plsc.VectorSubcoreMesh(core_axis_name="c", subcore_axis_name="s")
    f = functools.partial(
        pl.kernel,
        out_type=jax.ShapeDtypeStruct((B, D_EMBED), jnp.float32),
        mesh=mesh,
        scratch_types=[
            pltpu.VMEM((ROWS_PER_W, 128), jnp.int32),
            pltpu.VMEM((CHUNK, D_EMBED), jnp.float32),
            pltpu.VMEM((CHUNK, D_EMBED), jnp.float32),
            pltpu.SemaphoreType.DMA,
            pltpu.SemaphoreType.DMA,
            pltpu.SemaphoreType.DMA,
            pltpu.SemaphoreType.DMA,
        ],
        compiler_params=pltpu.CompilerParams(use_tc_tiling_on_sc=False),
    )(_gather_body)
    return f(x2d, table_lin)


def kernel(x, table):
    t2 = _relayout(table.T)
    table_lin = t2.reshape(4 * CSTRIDE, D_EMBED)
    q = ((x & (CSTRIDE - 1)) << 2) | lax.shift_right_logical(x, 18)
    x2d = q.reshape(IDX_ROWS, 128)
    out = _gather(x2d, table_lin)
    return out.reshape(BATCH, HIST, D_EMBED)


# speedup vs baseline: 1.1071x; 1.1071x over previous
import functools
import math

import jax
import jax.numpy as jnp
from jax import lax
from jax.experimental import pallas as pl
from jax.experimental.pallas import tpu as pltpu
from jax.experimental.pallas import tpu_sc as plsc

D_EMBED = 32
VOCAB = 1000000
BATCH = 16384
HIST = 20
SCALE = math.sqrt(D_EMBED)

NC, NS = 2, 16
NW = NC * NS
B = BATCH * HIST
IDX_ROWS = B // 128
ROWS_PER_W = IDX_ROWS // NW
CHUNK_ROWS = 8
CHUNK = CHUNK_ROWS * 128
NCHUNK = ROWS_PER_W // CHUNK_ROWS

CSTRIDE = 1 << 18
TCB = 8192
TGRID = CSTRIDE // TCB
LAST_BLK = (VOCAB - 1) // TCB


def _relayout_body(t0, t1, t2, t3, o_ref):
    eye_s = jnp.float32(SCALE) * jnp.eye(D_EMBED, dtype=jnp.float32)
    for c, t in enumerate((t0, t1, t2, t3)):
        o_ref[:, c * D_EMBED:(c + 1) * D_EMBED] = lax.dot_general(
            t[...], eye_s, (((0,), (0,)), ((), ())),
            preferred_element_type=jnp.float32)


def _relayout(table_t):
    def imap(c):
        return lambda i: (0, jnp.minimum(c * TGRID + i, LAST_BLK))
    return pl.pallas_call(
        _relayout_body,
        grid=(TGRID,),
        in_specs=[pl.BlockSpec((D_EMBED, TCB), imap(c)) for c in range(4)],
        out_specs=pl.BlockSpec((TCB, 4 * D_EMBED), lambda i: (i, 0)),
        out_shape=jax.ShapeDtypeStruct((CSTRIDE, 4 * D_EMBED), jnp.float32),
    )(table_t, table_t, table_t, table_t)


def _gather_body(x_hbm, table_hbm, out_hbm, idx_v, rows0, rows1, sg0, sg1,
                 ss0, ss1):
    wid = lax.axis_index("s") * NC + lax.axis_index("c")
    row_base = wid * ROWS_PER_W
    out_base = wid * ROWS_PER_W * 128

    pltpu.sync_copy(x_hbm.at[pl.ds(row_base, ROWS_PER_W)], idx_v)

    def fire(g, buf, sem):
        for j in range(CHUNK_ROWS):
            pltpu.async_copy(
                table_hbm.at[idx_v.at[g * CHUNK_ROWS + j]],
                buf.at[pl.ds(j * 128, 128)],
                sem,
            )

    def drain_gathers(buf, sem):
        for j in range(CHUNK_ROWS):
            pltpu.make_async_copy(
                table_hbm.at[idx_v.at[j]],
                buf.at[pl.ds(j * 128, 128)],
                sem,
            ).wait()

    def out_at(g):
        return out_hbm.at[pl.ds(out_base + g * CHUNK, CHUNK)]

    def store(g, buf, sem):
        pltpu.async_copy(buf, out_at(g), sem)

    def drain_store(g, buf, sem):
        pltpu.make_async_copy(buf, out_at(g), sem).wait()

    fire(0, rows0, sg0)
    fire(1, rows1, sg1)

    @pl.loop(0, NCHUNK // 2 - 1)
    def _pair(i):
        g = 2 * i
        drain_gathers(rows0, sg0)
        store(g, rows0, ss0)
        drain_gathers(rows1, sg1)
        store(g + 1, rows1, ss1)
        drain_store(g, rows0, ss0)
        fire(g + 2, rows0, sg0)
        drain_store(g + 1, rows1, ss1)
        fire(g + 3, rows1, sg1)

    last = NCHUNK - 2
    drain_gathers(rows0, sg0)
    store(last, rows0, ss0)
    drain_gathers(rows1, sg1)
    store(last + 1, rows1, ss1)
    drain_store(last, rows0, ss0)
    drain_store(last + 1, rows1, ss1)


def _gather(x2d, table_lin):
    mesh = plsc.VectorSubcoreMesh(core_axis_name="c", subcore_axis_name="s")
    f = functools.partial(
        pl.kernel,
        out_type=jax.ShapeDtypeStruct((B, D_EMBED), jnp.float32),
        mesh=mesh,
        scratch_types=[
            pltpu.VMEM((ROWS_PER_W, 128), jnp.int32),
            pltpu.VMEM((CHUNK, D_EMBED), jnp.float32),
            pltpu.VMEM((CHUNK, D_EMBED), jnp.float32),
            pltpu.SemaphoreType.DMA,
            pltpu.SemaphoreType.DMA,
            pltpu.SemaphoreType.DMA,
            pltpu.SemaphoreType.DMA,
        ],
        compiler_params=pltpu.CompilerParams(use_tc_tiling_on_sc=False),
    )(_gather_body)
    return f(x2d, table_lin)


def kernel(x, table):
    t2 = _relayout(table.T)
    table_lin = t2.reshape(4 * CSTRIDE, D_EMBED)
    q = ((x & (CSTRIDE - 1)) << 2) | lax.shift_right_logical(x, 18)
    x2d = q.reshape(IDX_ROWS, 128)
    out = _gather(x2d, table_lin)
    return out.reshape(BATCH, HIST, D_EMBED)
